# Initial kernel scaffold; baseline (speedup 1.0000x reference)
import jax
import jax.numpy as jnp
from jax.experimental import pallas as pl


def kernel(x, edge_index, W1, b1, Wmu, bmu, Wls, bls):
    raise NotImplementedError("write your pallas kernel here")


# speedup vs baseline: 20.0294x; 20.0294x over previous
import functools

import jax
import jax.numpy as jnp
from jax import lax
from jax.experimental import pallas as pl
from jax.experimental.pallas import tpu as pltpu
from jax.experimental.pallas import tpu_sc as plsc

N = 10000
E = 320000
D_IN = 128
D_OUT = 64
D_HID = 2 * D_OUT

NC = 2
NS = 16
NW = NC * NS
EPW = E // NW
CH = 80
NCH = EPW // CH
RPT = 624
TAIL0 = NS * RPT
TAILN = N - TAIL0

BN = 1000
GRID = N // BN

@functools.cache
def _sc_kernels():
    mesh = plsc.VectorSubcoreMesh(core_axis_name="c", subcore_axis_name="s")

    @functools.partial(
        pl.kernel,
        mesh=mesh,
        out_type=jax.ShapeDtypeStruct((NC, N, 1), jnp.float32),
        scratch_types=[
            pltpu.VMEM((NCH, CH), jnp.int32),
            pltpu.VMEM((CH, 1), jnp.float32),
            pltpu.VMEM_SHARED((N, 1), jnp.float32),
        ],
    )
    def sc_degree(dst_hbm, ones_hbm, out_hbm, dst_v, ones_v, acc):
        cid = lax.axis_index("c")
        sid = lax.axis_index("s")
        wid = cid * NS + sid
        pltpu.sync_copy(ones_hbm.at[pl.ds(sid * RPT, RPT)], acc.at[pl.ds(sid * RPT, RPT)])

        @pl.when(sid == NS - 1)
        def _():
            pltpu.sync_copy(ones_hbm.at[pl.ds(TAIL0, TAILN)], acc.at[pl.ds(TAIL0, TAILN)])

        pltpu.sync_copy(ones_hbm.at[pl.ds(0, CH)], ones_v)
        pltpu.sync_copy(dst_hbm.at[wid], dst_v)
        plsc.subcore_barrier()

        def body(j, carry):
            pltpu.sync_copy(ones_v, acc.at[dst_v.at[j]], add=True)
            return carry

        lax.fori_loop(0, NCH, body, 0)
        plsc.subcore_barrier()
        pltpu.sync_copy(acc.at[pl.ds(sid * RPT, RPT)], out_hbm.at[cid, pl.ds(sid * RPT, RPT)])

        @pl.when(sid == NS - 1)
        def _():
            pltpu.sync_copy(acc.at[pl.ds(TAIL0, TAILN)], out_hbm.at[cid, pl.ds(TAIL0, TAILN)])

    @functools.partial(
        pl.kernel,
        mesh=mesh,
        out_type=jax.ShapeDtypeStruct((NC, N, D_IN), jnp.float32),
        scratch_types=[
            pltpu.VMEM((NCH, CH), jnp.int32),
            pltpu.VMEM((NCH, CH), jnp.int32),
            pltpu.VMEM((CH, D_IN), jnp.float32),
            pltpu.VMEM_SHARED((N, D_IN), jnp.float32),
            pltpu.SemaphoreType.DMA,
        ],
    )
    def sc_aggregate(feat_hbm, src_hbm, dst_hbm, out_hbm, src_v, dst_v, rows_v, acc, sem):
        cid = lax.axis_index("c")
        sid = lax.axis_index("s")
        wid = cid * NS + sid
        r0 = sid * RPT
        pltpu.sync_copy(feat_hbm.at[pl.ds(r0, RPT)], acc.at[pl.ds(r0, RPT)])

        @pl.when(sid == NS - 1)
        def _():
            pltpu.sync_copy(feat_hbm.at[pl.ds(TAIL0, TAILN)], acc.at[pl.ds(TAIL0, TAILN)])

        pltpu.sync_copy(src_hbm.at[wid], src_v)
        pltpu.sync_copy(dst_hbm.at[wid], dst_v)
        plsc.subcore_barrier()

        def body(j, carry):
            pltpu.async_copy(feat_hbm.at[src_v.at[j]], rows_v, sem).wait()
            pltpu.sync_copy(rows_v, acc.at[dst_v.at[j]], add=True)
            return carry

        lax.fori_loop(0, NCH, body, 0)
        plsc.subcore_barrier()
        pltpu.sync_copy(acc.at[pl.ds(r0, RPT)], out_hbm.at[cid, pl.ds(r0, RPT)])

        @pl.when(sid == NS - 1)
        def _():
            pltpu.sync_copy(acc.at[pl.ds(TAIL0, TAILN)], out_hbm.at[cid, pl.ds(TAIL0, TAILN)])

    return sc_degree, sc_aggregate


def _tc_scale_body(deg_ref, x_ref, xs_ref, dis_ref):
    deg = deg_ref[:, 0:1] + deg_ref[:, 1:2] - 1.0
    dis = lax.rsqrt(deg)
    dis_ref[...] = dis
    xs_ref[...] = x_ref[...] * dis


def _tc_mid_body(dis_ref, xs_ref, a0_ref, a1_ref, w1_ref, b1_ref, wc_ref, ts_ref):
    dis = dis_ref[...]
    pre = (a0_ref[...] + a1_ref[...] - xs_ref[...]) * dis
    h = jnp.maximum(
        jnp.dot(pre, w1_ref[...], preferred_element_type=jnp.float32) + b1_ref[...],
        0.0,
    )
    t = jnp.dot(h, wc_ref[...], preferred_element_type=jnp.float32)
    ts_ref[...] = t * dis


def _tc_out_body(dis_ref, ts_ref, a0_ref, a1_ref, bc_ref, o_ref):
    o_ref[...] = (a0_ref[...] + a1_ref[...] - ts_ref[...]) * dis_ref[...] + bc_ref[...]


def _row_spec(d):
    return pl.BlockSpec((BN, d), lambda i: (i, 0))


def _full_spec(r, c):
    return pl.BlockSpec((r, c), lambda i: (0, 0))


def kernel(x, edge_index, W1, b1, Wmu, bmu, Wls, bls):
    src = edge_index[0].reshape(NW, NCH, CH)
    dst = edge_index[1].reshape(NW, NCH, CH)
    ones_col = jnp.ones((N, 1), jnp.float32)
    Wcat = jnp.concatenate([Wmu, Wls], axis=1)
    bcat = jnp.concatenate([bmu, bls])[None, :]
    b1r = b1[None, :]

    sc_degree, sc_aggregate = _sc_kernels()
    deg_p = sc_degree(dst, ones_col)
    degT = jnp.transpose(deg_p[:, :, 0])

    xs, dis = pl.pallas_call(
        _tc_scale_body,
        grid=(GRID,),
        in_specs=[_row_spec(2), _row_spec(D_IN)],
        out_specs=[_row_spec(D_IN), _row_spec(1)],
        out_shape=[
            jax.ShapeDtypeStruct((N, D_IN), jnp.float32),
            jax.ShapeDtypeStruct((N, 1), jnp.float32),
        ],
    )(degT, x)

    a1 = sc_aggregate(xs, src, dst)

    ts = pl.pallas_call(
        _tc_mid_body,
        grid=(GRID,),
        in_specs=[
            _row_spec(1), _row_spec(D_IN), _row_spec(D_IN), _row_spec(D_IN),
            _full_spec(D_IN, D_HID), _full_spec(1, D_HID), _full_spec(D_HID, 2 * D_OUT),
        ],
        out_specs=_row_spec(D_IN),
        out_shape=jax.ShapeDtypeStruct((N, D_IN), jnp.float32),
    )(dis, xs, a1[0], a1[1], W1, b1r, Wcat)

    a2 = sc_aggregate(ts, src, dst)

    out = pl.pallas_call(
        _tc_out_body,
        grid=(GRID,),
        in_specs=[
            _row_spec(1), _row_spec(D_IN), _row_spec(D_IN), _row_spec(D_IN),
            _full_spec(1, 2 * D_OUT),
        ],
        out_specs=_row_spec(D_IN),
        out_shape=jax.ShapeDtypeStruct((N, D_IN), jnp.float32),
    )(dis, ts, a2[0], a2[1], bcat)

    return (out[:, :D_OUT], out[:, D_OUT:])


# speedup vs baseline: 23.9725x; 1.1969x over previous
import functools

import jax
import jax.numpy as jnp
from jax import lax
from jax.experimental import pallas as pl
from jax.experimental.pallas import tpu as pltpu
from jax.experimental.pallas import tpu_sc as plsc

N = 10000
E = 320000
D_IN = 128
D_OUT = 64
D_HID = 2 * D_OUT

NC = 2
NS = 16
NW = NC * NS
EPW = E // NW
CH = 80
NCH = EPW // CH
SEG = 25
RPT = 624
TAIL0 = NS * RPT
TAILN = N - TAIL0

BN = 1000
GRID = N // BN

@functools.cache
def _sc_kernels():
    mesh = plsc.VectorSubcoreMesh(core_axis_name="c", subcore_axis_name="s")

    @functools.partial(
        pl.kernel,
        mesh=mesh,
        out_type=jax.ShapeDtypeStruct((NC, N, 1), jnp.float32),
        scratch_types=[
            pltpu.VMEM((NCH, CH), jnp.int32),
            pltpu.VMEM((CH, 1), jnp.float32),
            pltpu.VMEM_SHARED((N, 1), jnp.float32),
        ],
    )
    def sc_degree(dst_hbm, ones_hbm, out_hbm, dst_v, ones_v, acc):
        cid = lax.axis_index("c")
        sid = lax.axis_index("s")
        wid = cid * NS + sid
        pltpu.sync_copy(ones_hbm.at[pl.ds(sid * RPT, RPT)], acc.at[pl.ds(sid * RPT, RPT)])

        @pl.when(sid == NS - 1)
        def _():
            pltpu.sync_copy(ones_hbm.at[pl.ds(TAIL0, TAILN)], acc.at[pl.ds(TAIL0, TAILN)])

        pltpu.sync_copy(ones_hbm.at[pl.ds(0, CH)], ones_v)
        pltpu.sync_copy(dst_hbm.at[wid], dst_v)
        plsc.subcore_barrier()

        def body(j, carry):
            pltpu.sync_copy(ones_v, acc.at[dst_v.at[j]], add=True)
            return carry

        lax.fori_loop(0, NCH, body, 0)
        plsc.subcore_barrier()
        pltpu.sync_copy(acc.at[pl.ds(sid * RPT, RPT)], out_hbm.at[cid, pl.ds(sid * RPT, RPT)])

        @pl.when(sid == NS - 1)
        def _():
            pltpu.sync_copy(acc.at[pl.ds(TAIL0, TAILN)], out_hbm.at[cid, pl.ds(TAIL0, TAILN)])

    @functools.partial(
        pl.kernel,
        mesh=mesh,
        out_type=jax.ShapeDtypeStruct((NC, N, D_IN), jnp.float32),
        scratch_types=[
            pltpu.VMEM((NCH, CH), jnp.int32),
            pltpu.VMEM((SEG, CH), jnp.int32),
            pltpu.VMEM((CH, D_IN), jnp.float32),
            pltpu.VMEM((CH, D_IN), jnp.float32),
            pltpu.VMEM_SHARED((N, D_IN), jnp.float32),
            pltpu.SemaphoreType.DMA,
            pltpu.SemaphoreType.DMA,
        ],
    )
    def sc_aggregate(feat_hbm, src_hbm, dst_hbm, out_hbm, src_v, dst_v, b0, b1, acc, sem0, sem1):
        cid = lax.axis_index("c")
        sid = lax.axis_index("s")
        wid = cid * NS + sid
        r0 = sid * RPT
        pltpu.sync_copy(feat_hbm.at[pl.ds(r0, RPT)], acc.at[pl.ds(r0, RPT)])

        @pl.when(sid == NS - 1)
        def _():
            pltpu.sync_copy(feat_hbm.at[pl.ds(TAIL0, TAILN)], acc.at[pl.ds(TAIL0, TAILN)])

        pltpu.sync_copy(src_hbm.at[wid], src_v)
        pltpu.sync_copy(dst_hbm.at[wid, 0], dst_v)
        plsc.subcore_barrier()

        def g_start(j, buf, sem):
            pltpu.make_async_copy(feat_hbm.at[src_v.at[j]], buf, sem).start()

        def g_wait(j, buf, sem):
            pltpu.make_async_copy(feat_hbm.at[src_v.at[j]], buf, sem).wait()

        def dst_reload(j):
            @pl.when(lax.rem(j, SEG) == 0)
            def _():
                pltpu.sync_copy(dst_hbm.at[wid, lax.div(j, SEG)], dst_v)

        def scat(j, buf):
            pltpu.sync_copy(buf, acc.at[dst_v.at[lax.rem(j, SEG)]], add=True)

        g_start(0, b0, sem0)

        def body(jj, carry):
            j = 2 * jj
            g_wait(j, b0, sem0)
            g_start(j + 1, b1, sem1)
            dst_reload(j)
            scat(j, b0)
            g_wait(j + 1, b1, sem1)
            g_start(j + 2, b0, sem0)
            dst_reload(j + 1)
            scat(j + 1, b1)
            return carry

        lax.fori_loop(0, (NCH - 1) // 2, body, 0)
        g_wait(NCH - 1, b0, sem0)
        scat(NCH - 1, b0)
        plsc.subcore_barrier()
        pltpu.sync_copy(acc.at[pl.ds(r0, RPT)], out_hbm.at[cid, pl.ds(r0, RPT)])

        @pl.when(sid == NS - 1)
        def _():
            pltpu.sync_copy(acc.at[pl.ds(TAIL0, TAILN)], out_hbm.at[cid, pl.ds(TAIL0, TAILN)])

    return sc_degree, sc_aggregate


def _tc_scale_body(deg_ref, x_ref, xs_ref, dis_ref):
    deg = deg_ref[:, 0:1] + deg_ref[:, 1:2] - 1.0
    dis = lax.rsqrt(deg)
    dis_ref[...] = dis
    xs_ref[...] = x_ref[...] * dis


def _tc_mid_body(dis_ref, xs_ref, a0_ref, a1_ref, w1_ref, b1_ref, wc_ref, ts_ref):
    dis = dis_ref[...]
    pre = (a0_ref[...] + a1_ref[...] - xs_ref[...]) * dis
    h = jnp.maximum(
        jnp.dot(pre, w1_ref[...], preferred_element_type=jnp.float32) + b1_ref[...],
        0.0,
    )
    t = jnp.dot(h, wc_ref[...], preferred_element_type=jnp.float32)
    ts_ref[...] = t * dis


def _tc_out_body(dis_ref, ts_ref, a0_ref, a1_ref, bc_ref, o_ref):
    o_ref[...] = (a0_ref[...] + a1_ref[...] - ts_ref[...]) * dis_ref[...] + bc_ref[...]


def _row_spec(d):
    return pl.BlockSpec((BN, d), lambda i: (i, 0))


def _full_spec(r, c):
    return pl.BlockSpec((r, c), lambda i: (0, 0))


def kernel(x, edge_index, W1, b1, Wmu, bmu, Wls, bls):
    src = edge_index[0].reshape(NW, NCH, CH)
    dst = edge_index[1].reshape(NW, NCH, CH)
    dst4 = edge_index[1].reshape(NW, NCH // SEG, SEG, CH)
    ones_col = jnp.ones((N, 1), jnp.float32)
    Wcat = jnp.concatenate([Wmu, Wls], axis=1)
    bcat = jnp.concatenate([bmu, bls])[None, :]
    b1r = b1[None, :]

    sc_degree, sc_aggregate = _sc_kernels()
    deg_p = sc_degree(dst, ones_col)
    degT = jnp.transpose(deg_p[:, :, 0])

    xs, dis = pl.pallas_call(
        _tc_scale_body,
        grid=(GRID,),
        in_specs=[_row_spec(2), _row_spec(D_IN)],
        out_specs=[_row_spec(D_IN), _row_spec(1)],
        out_shape=[
            jax.ShapeDtypeStruct((N, D_IN), jnp.float32),
            jax.ShapeDtypeStruct((N, 1), jnp.float32),
        ],
    )(degT, x)

    a1 = sc_aggregate(xs, src, dst4)

    ts = pl.pallas_call(
        _tc_mid_body,
        grid=(GRID,),
        in_specs=[
            _row_spec(1), _row_spec(D_IN), _row_spec(D_IN), _row_spec(D_IN),
            _full_spec(D_IN, D_HID), _full_spec(1, D_HID), _full_spec(D_HID, 2 * D_OUT),
        ],
        out_specs=_row_spec(D_IN),
        out_shape=jax.ShapeDtypeStruct((N, D_IN), jnp.float32),
    )(dis, xs, a1[0], a1[1], W1, b1r, Wcat)

    a2 = sc_aggregate(ts, src, dst4)

    out = pl.pallas_call(
        _tc_out_body,
        grid=(GRID,),
        in_specs=[
            _row_spec(1), _row_spec(D_IN), _row_spec(D_IN), _row_spec(D_IN),
            _full_spec(1, 2 * D_OUT),
        ],
        out_specs=_row_spec(D_IN),
        out_shape=jax.ShapeDtypeStruct((N, D_IN), jnp.float32),
    )(dis, ts, a2[0], a2[1], bcat)

    return (out[:, :D_OUT], out[:, D_OUT:])


# speedup vs baseline: 24.1515x; 1.0075x over previous
import functools

import jax
import jax.numpy as jnp
from jax import lax
from jax.experimental import pallas as pl
from jax.experimental.pallas import tpu as pltpu
from jax.experimental.pallas import tpu_sc as plsc

N = 10000
E = 320000
D_IN = 128
D_OUT = 64
D_HID = 2 * D_OUT

NC = 2
NS = 16
NW = NC * NS
EPW = E // NW
CH = 80
NCH = EPW // CH
SEG = 25
RPT = 624
TAIL0 = NS * RPT
TAILN = N - TAIL0

BN = 1000
GRID = N // BN

@functools.cache
def _sc_kernels():
    mesh = plsc.VectorSubcoreMesh(core_axis_name="c", subcore_axis_name="s")

    @functools.partial(
        pl.kernel,
        mesh=mesh,
        out_type=jax.ShapeDtypeStruct((NC, N, 1), jnp.float32),
        scratch_types=[
            pltpu.VMEM((NCH, CH), jnp.int32),
            pltpu.VMEM((CH, 1), jnp.float32),
            pltpu.VMEM_SHARED((N, 1), jnp.float32),
        ],
    )
    def sc_degree(dst_hbm, ones_hbm, out_hbm, dst_v, ones_v, acc):
        cid = lax.axis_index("c")
        sid = lax.axis_index("s")
        wid = cid * NS + sid
        pltpu.sync_copy(ones_hbm.at[pl.ds(sid * RPT, RPT)], acc.at[pl.ds(sid * RPT, RPT)])

        @pl.when(sid == NS - 1)
        def _():
            pltpu.sync_copy(ones_hbm.at[pl.ds(TAIL0, TAILN)], acc.at[pl.ds(TAIL0, TAILN)])

        pltpu.sync_copy(ones_hbm.at[pl.ds(0, CH)], ones_v)
        pltpu.sync_copy(dst_hbm.at[wid], dst_v)
        plsc.subcore_barrier()

        def body(j, carry):
            pltpu.sync_copy(ones_v, acc.at[dst_v.at[j]], add=True)
            return carry

        lax.fori_loop(0, NCH, body, 0)
        plsc.subcore_barrier()
        pltpu.sync_copy(acc.at[pl.ds(sid * RPT, RPT)], out_hbm.at[cid, pl.ds(sid * RPT, RPT)])

        @pl.when(sid == NS - 1)
        def _():
            pltpu.sync_copy(acc.at[pl.ds(TAIL0, TAILN)], out_hbm.at[cid, pl.ds(TAIL0, TAILN)])

    @functools.partial(
        pl.kernel,
        mesh=mesh,
        out_type=jax.ShapeDtypeStruct((NC, N, D_IN), jnp.float32),
        scratch_types=[
            pltpu.VMEM((NCH, CH), jnp.int32),
            pltpu.VMEM((2, SEG, CH), jnp.int32),
            pltpu.VMEM((CH, D_IN), jnp.float32),
            pltpu.VMEM((CH, D_IN), jnp.float32),
            pltpu.VMEM_SHARED((N, D_IN), jnp.float32),
            pltpu.SemaphoreType.DMA,
            pltpu.SemaphoreType.DMA,
            pltpu.SemaphoreType.DMA,
            pltpu.SemaphoreType.DMA,
        ],
    )
    def sc_aggregate(feat_hbm, src_hbm, dst_hbm, out_hbm,
                     src_v, dst_v, b0, b1, acc, gs0, gs1, ss0, ss1):
        cid = lax.axis_index("c")
        sid = lax.axis_index("s")
        wid = cid * NS + sid
        r0 = sid * RPT
        pltpu.sync_copy(feat_hbm.at[pl.ds(r0, RPT)], acc.at[pl.ds(r0, RPT)])

        @pl.when(sid == NS - 1)
        def _():
            pltpu.sync_copy(feat_hbm.at[pl.ds(TAIL0, TAILN)], acc.at[pl.ds(TAIL0, TAILN)])

        pltpu.sync_copy(src_hbm.at[wid], src_v)
        pltpu.sync_copy(dst_hbm.at[wid, 0], dst_v.at[0])
        plsc.subcore_barrier()

        def g_start(j, buf, sem):
            pltpu.make_async_copy(feat_hbm.at[src_v.at[j]], buf, sem).start()

        def g_wait(j, buf, sem):
            pltpu.make_async_copy(feat_hbm.at[src_v.at[j]], buf, sem).wait()

        def dst_row(j):
            seg = lax.div(j, SEG)
            return dst_v.at[lax.rem(seg, 2), lax.rem(j, SEG)]

        def dst_reload(j):
            @pl.when(lax.rem(j, SEG) == 0)
            def _():
                seg = lax.div(j, SEG)
                pltpu.sync_copy(dst_hbm.at[wid, seg], dst_v.at[lax.rem(seg, 2)])

        def s_start(j, buf, sem):
            pltpu.async_copy(buf, acc.at[dst_row(j)], sem, add=True)

        def s_wait(j, buf, sem):
            pltpu.make_async_copy(buf, acc.at[dst_row(j)], sem).wait()

        g_start(0, b0, gs0)
        g_start(1, b1, gs1)

        def body(jj, carry):
            j = 2 * jj
            g_wait(j, b0, gs0)
            dst_reload(j)
            s_start(j, b0, ss0)
            g_wait(j + 1, b1, gs1)
            dst_reload(j + 1)
            s_start(j + 1, b1, ss1)

            @pl.when(j + 2 < NCH)
            def _():
                s_wait(j, b0, ss0)
                g_start(j + 2, b0, gs0)

            @pl.when(j + 3 < NCH)
            def _():
                s_wait(j + 1, b1, ss1)
                g_start(j + 3, b1, gs1)

            return carry

        lax.fori_loop(0, NCH // 2, body, 0)
        g_wait(NCH - 1, b0, gs0)
        s_start(NCH - 1, b0, ss0)
        s_wait(NCH - 2, b1, ss1)
        s_wait(NCH - 1, b0, ss0)
        plsc.subcore_barrier()
        pltpu.sync_copy(acc.at[pl.ds(r0, RPT)], out_hbm.at[cid, pl.ds(r0, RPT)])

        @pl.when(sid == NS - 1)
        def _():
            pltpu.sync_copy(acc.at[pl.ds(TAIL0, TAILN)], out_hbm.at[cid, pl.ds(TAIL0, TAILN)])

    return sc_degree, sc_aggregate


def _tc_scale_body(deg_ref, x_ref, xs_ref, dis_ref):
    deg = deg_ref[:, 0:1] + deg_ref[:, 1:2] - 1.0
    dis = lax.rsqrt(deg)
    dis_ref[...] = dis
    xs_ref[...] = x_ref[...] * dis


def _tc_mid_body(dis_ref, xs_ref, a0_ref, a1_ref, w1_ref, b1_ref, wc_ref, ts_ref):
    dis = dis_ref[...]
    pre = (a0_ref[...] + a1_ref[...] - xs_ref[...]) * dis
    h = jnp.maximum(
        jnp.dot(pre, w1_ref[...], preferred_element_type=jnp.float32) + b1_ref[...],
        0.0,
    )
    t = jnp.dot(h, wc_ref[...], preferred_element_type=jnp.float32)
    ts_ref[...] = t * dis


def _tc_out_body(dis_ref, ts_ref, a0_ref, a1_ref, bc_ref, o_ref):
    o_ref[...] = (a0_ref[...] + a1_ref[...] - ts_ref[...]) * dis_ref[...] + bc_ref[...]


def _row_spec(d):
    return pl.BlockSpec((BN, d), lambda i: (i, 0))


def _full_spec(r, c):
    return pl.BlockSpec((r, c), lambda i: (0, 0))


def kernel(x, edge_index, W1, b1, Wmu, bmu, Wls, bls):
    src = edge_index[0].reshape(NW, NCH, CH)
    dst = edge_index[1].reshape(NW, NCH, CH)
    dst4 = edge_index[1].reshape(NW, NCH // SEG, SEG, CH)
    ones_col = jnp.ones((N, 1), jnp.float32)
    Wcat = jnp.concatenate([Wmu, Wls], axis=1)
    bcat = jnp.concatenate([bmu, bls])[None, :]
    b1r = b1[None, :]

    sc_degree, sc_aggregate = _sc_kernels()
    deg_p = sc_degree(dst, ones_col)
    degT = jnp.transpose(deg_p[:, :, 0])

    xs, dis = pl.pallas_call(
        _tc_scale_body,
        grid=(GRID,),
        in_specs=[_row_spec(2), _row_spec(D_IN)],
        out_specs=[_row_spec(D_IN), _row_spec(1)],
        out_shape=[
            jax.ShapeDtypeStruct((N, D_IN), jnp.float32),
            jax.ShapeDtypeStruct((N, 1), jnp.float32),
        ],
    )(degT, x)

    a1 = sc_aggregate(xs, src, dst4)

    ts = pl.pallas_call(
        _tc_mid_body,
        grid=(GRID,),
        in_specs=[
            _row_spec(1), _row_spec(D_IN), _row_spec(D_IN), _row_spec(D_IN),
            _full_spec(D_IN, D_HID), _full_spec(1, D_HID), _full_spec(D_HID, 2 * D_OUT),
        ],
        out_specs=_row_spec(D_IN),
        out_shape=jax.ShapeDtypeStruct((N, D_IN), jnp.float32),
    )(dis, xs, a1[0], a1[1], W1, b1r, Wcat)

    a2 = sc_aggregate(ts, src, dst4)

    out = pl.pallas_call(
        _tc_out_body,
        grid=(GRID,),
        in_specs=[
            _row_spec(1), _row_spec(D_IN), _row_spec(D_IN), _row_spec(D_IN),
            _full_spec(1, 2 * D_OUT),
        ],
        out_specs=_row_spec(D_IN),
        out_shape=jax.ShapeDtypeStruct((N, D_IN), jnp.float32),
    )(dis, ts, a2[0], a2[1], bcat)

    return (out[:, :D_OUT], out[:, D_OUT:])
